# baseline (speedup 1.0000x reference)
import jax
import jax.numpy as jnp
from jax import lax
from jax.experimental import pallas as pl
from jax.experimental.pallas import tpu as pltpu
from jax.experimental.pallas import tpu_sc as plsc

N = 10000
E = 320000
F_IN = 128
DP = 40
ONES_COL = 36

NC = 2
NS = 16
NW = NC * NS
CHUNK = 125
NCHUNKS = E // CHUNK
CH_PER_W = NCHUNKS // NW
KSLOT = 16
AHEAD = 8
N_PAD = 10240
ROWS_PER_TILE = N_PAD // NS


def _sc_agg_body(p_hbm, src_hbm, dst_hbm, zeros_hbm, out_hbm,
                 sidx, didx, rows, agg, sem_g, sem_s, sem_i):
    c = lax.axis_index("c")
    s = lax.axis_index("s")
    w = s * NC + c

    hi1 = pltpu.async_copy(src_hbm.at[pl.ds(w * CH_PER_W, CH_PER_W)], sidx, sem_i)
    hi2 = pltpu.async_copy(dst_hbm.at[pl.ds(w * CH_PER_W, CH_PER_W)], didx, sem_i)
    pltpu.sync_copy(zeros_hbm.at[pl.ds(s * ROWS_PER_TILE, ROWS_PER_TILE)],
                    agg.at[pl.ds(s * ROWS_PER_TILE, ROWS_PER_TILE)])
    hi1.wait()
    hi2.wait()

    def _fire(slot, chunk):
        pltpu.async_copy(p_hbm.at[sidx.at[chunk]], rows.at[slot], sem_g)

    def _wait_gather(slot, chunk):
        pltpu.make_async_copy(p_hbm.at[sidx.at[chunk]], rows.at[slot],
                              sem_g).wait()

    def _scatter(slot, chunk):
        pltpu.async_copy(rows.at[slot], agg.at[didx.at[chunk]], sem_s, add=True)

    def _wait_scatter(slot, chunk):
        pltpu.make_async_copy(rows.at[slot], agg.at[didx.at[chunk]],
                              sem_s).wait()

    for b in range(AHEAD):
        _fire(b, b)
    plsc.subcore_barrier()

    def _group(g, carry):
        i0 = g * KSLOT
        for b in range(KSLOT):
            chunk = i0 + b
            _wait_gather(b, chunk)
            _scatter(b, chunk)
            prev = chunk - AHEAD

            @pl.when(prev >= 0)
            def _():
                _wait_scatter((b - AHEAD) % KSLOT, prev)
            nxt = chunk + AHEAD

            @pl.when(nxt < CH_PER_W)
            def _():
                _fire((b + AHEAD) % KSLOT, nxt)
        return carry

    lax.fori_loop(0, CH_PER_W // KSLOT, _group, 0)

    for b in range(AHEAD):
        chunk = CH_PER_W - AHEAD + b
        _wait_scatter(chunk % KSLOT, chunk)

    plsc.subcore_barrier()
    pltpu.sync_copy(agg.at[pl.ds(s * ROWS_PER_TILE, ROWS_PER_TILE)],
                    out_hbm.at[c, pl.ds(s * ROWS_PER_TILE, ROWS_PER_TILE)])


_sc_agg = pl.kernel(
    _sc_agg_body,
    out_type=jax.ShapeDtypeStruct((NC, N_PAD, DP), jnp.float32),
    mesh=plsc.VectorSubcoreMesh(core_axis_name="c", subcore_axis_name="s"),
    scratch_types=[
        pltpu.VMEM((CH_PER_W, CHUNK), jnp.int32),
        pltpu.VMEM((CH_PER_W, CHUNK), jnp.int32),
        pltpu.VMEM((KSLOT, CHUNK, DP), jnp.float32),
        pltpu.VMEM_SHARED((N_PAD, DP), jnp.float32),
        pltpu.SemaphoreType.DMA,
        pltpu.SemaphoreType.DMA,
        pltpu.SemaphoreType.DMA,
    ],
    compiler_params=pltpu.CompilerParams(use_tc_tiling_on_sc=False),
)


def _tc_a_body(x_ref, w1l_ref, w1r_ref, p1_hbm, r1_ref, vm, sem):
    x = x_ref[...]
    xw = jnp.dot(x, w1l_ref[...], preferred_element_type=jnp.float32)
    col = lax.broadcasted_iota(jnp.int32, (N, DP), 1)
    vm[...] = jnp.where(col == ONES_COL, 1.0, xw)
    cp = pltpu.make_async_copy(vm, p1_hbm, sem)
    cp.start()
    r1_ref[...] = jnp.dot(x, w1r_ref[...], preferred_element_type=jnp.float32)
    cp.wait()


def _tc_b_body(parts_hbm, r1_ref, b1_ref, w2l_ref, w2r_ref, e36_ref,
               p2_hbm, r2_ref, deg_ref, pvm, vm, sem):
    pltpu.sync_copy(parts_hbm, pvm)
    agg = pvm[0, :N] + pvm[1, :N]
    deg = jnp.maximum(
        jnp.dot(agg, e36_ref[...], preferred_element_type=jnp.float32), 1.0)
    h = jnp.maximum(agg / deg + r1_ref[...] + b1_ref[...], 0.0)
    vm[...] = jnp.dot(h, w2l_ref[...], preferred_element_type=jnp.float32)
    cp = pltpu.make_async_copy(vm, p2_hbm, sem)
    cp.start()
    r2_ref[...] = jnp.dot(h, w2r_ref[...], preferred_element_type=jnp.float32)
    deg_ref[...] = deg
    cp.wait()


def _tc_c_body(parts_hbm, r2_ref, deg_ref, b2_ref, out_ref, pvm):
    pltpu.sync_copy(parts_hbm, pvm)
    agg = pvm[0, :N] + pvm[1, :N]
    out_ref[...] = agg / deg_ref[...] + r2_ref[...] + b2_ref[...]


_tc_a = pl.pallas_call(
    _tc_a_body,
    out_shape=[jax.ShapeDtypeStruct((N, DP), jnp.float32),
               jax.ShapeDtypeStruct((N, DP), jnp.float32)],
    out_specs=[pl.BlockSpec(memory_space=pl.ANY),
               pl.BlockSpec(memory_space=pltpu.MemorySpace.VMEM)],
    scratch_shapes=[pltpu.VMEM((N, DP), jnp.float32),
                    pltpu.SemaphoreType.DMA],
)

_tc_b = pl.pallas_call(
    _tc_b_body,
    in_specs=[pl.BlockSpec(memory_space=pl.ANY),
              pl.BlockSpec(memory_space=pltpu.MemorySpace.VMEM),
              pl.BlockSpec(memory_space=pltpu.MemorySpace.VMEM),
              pl.BlockSpec(memory_space=pltpu.MemorySpace.VMEM),
              pl.BlockSpec(memory_space=pltpu.MemorySpace.VMEM),
              pl.BlockSpec(memory_space=pltpu.MemorySpace.VMEM)],
    out_shape=[jax.ShapeDtypeStruct((N, DP), jnp.float32),
               jax.ShapeDtypeStruct((N, DP), jnp.float32),
               jax.ShapeDtypeStruct((N, 1), jnp.float32)],
    out_specs=[pl.BlockSpec(memory_space=pl.ANY),
               pl.BlockSpec(memory_space=pltpu.MemorySpace.VMEM),
               pl.BlockSpec(memory_space=pltpu.MemorySpace.VMEM)],
    scratch_shapes=[pltpu.VMEM((NC, N_PAD, DP), jnp.float32),
                    pltpu.VMEM((N, DP), jnp.float32),
                    pltpu.SemaphoreType.DMA],
)

_tc_c = pl.pallas_call(
    _tc_c_body,
    in_specs=[pl.BlockSpec(memory_space=pl.ANY),
              pl.BlockSpec(memory_space=pltpu.MemorySpace.VMEM),
              pl.BlockSpec(memory_space=pltpu.MemorySpace.VMEM),
              pl.BlockSpec(memory_space=pltpu.MemorySpace.VMEM)],
    out_shape=jax.ShapeDtypeStruct((N, DP), jnp.float32),
    scratch_shapes=[pltpu.VMEM((NC, N_PAD, DP), jnp.float32)],
)


def kernel(x, edge_index, W1_l, b1, W1_r, W2_l, b2, W2_r):
    src = edge_index[0].reshape(NCHUNKS, CHUNK)
    dst = edge_index[1].reshape(NCHUNKS, CHUNK)

    w1l = jnp.zeros((F_IN, DP), jnp.float32).at[:, :36].set(W1_l)
    w1r = jnp.zeros((F_IN, DP), jnp.float32).at[:, :36].set(W1_r)
    b1p = jnp.zeros((1, DP), jnp.float32).at[0, :36].set(b1)
    w2l = jnp.zeros((DP, DP), jnp.float32).at[:36, :40].set(W2_l)
    w2r = jnp.zeros((DP, DP), jnp.float32).at[:36, :40].set(W2_r)
    b2p = jnp.zeros((1, DP), jnp.float32).at[0, :40].set(b2)
    e36 = jnp.zeros((DP, 1), jnp.float32).at[ONES_COL, 0].set(1.0)
    zeros = jnp.zeros((N_PAD, DP), jnp.float32)

    p1, r1 = _tc_a(x, w1l, w1r)
    parts1 = _sc_agg(p1, src, dst, zeros)
    p2, r2, deg = _tc_b(parts1, r1, b1p, w2l, w2r, e36)
    parts2 = _sc_agg(p2, src, dst, zeros)
    return _tc_c(parts2, r2, deg, b2p)


# speedup vs baseline: 1.0237x; 1.0237x over previous
import jax
import jax.numpy as jnp
from jax import lax
from jax.experimental import pallas as pl
from jax.experimental.pallas import tpu as pltpu
from jax.experimental.pallas import tpu_sc as plsc

N = 10000
E = 320000
F_IN = 128
DP = 40
ONES_COL = 36

NC = 2
NS = 16
NW = NC * NS
CHUNK = 125
NCHUNKS = E // CHUNK
CH_PER_W = NCHUNKS // NW
KSLOT = 16
AHEAD = 8
N_PAD = 10240
ROWS_PER_TILE = N_PAD // NS


def _sc_agg_body(p_hbm, src_hbm, dst_hbm, zeros_hbm, out_hbm,
                 sidx, didx, rows, agg, sem_g, sem_s, sem_i):
    c = lax.axis_index("c")
    s = lax.axis_index("s")
    w = s * NC + c

    hi1 = pltpu.async_copy(src_hbm.at[pl.ds(w * CH_PER_W, CH_PER_W)], sidx, sem_i)
    hi2 = pltpu.async_copy(dst_hbm.at[pl.ds(w * CH_PER_W, CH_PER_W)], didx, sem_i)
    pltpu.sync_copy(zeros_hbm.at[pl.ds(s * ROWS_PER_TILE, ROWS_PER_TILE)],
                    agg.at[pl.ds(s * ROWS_PER_TILE, ROWS_PER_TILE)])
    hi1.wait()
    hi2.wait()

    def _fire(slot, chunk):
        pltpu.async_copy(p_hbm.at[sidx.at[chunk]], rows.at[slot], sem_g)

    def _wait_gather(slot, chunk):
        pltpu.make_async_copy(p_hbm.at[sidx.at[chunk]], rows.at[slot],
                              sem_g).wait()

    def _scatter(slot, chunk):
        pltpu.async_copy(rows.at[slot], agg.at[didx.at[chunk]], sem_s, add=True)

    def _wait_scatter(slot, chunk):
        pltpu.make_async_copy(rows.at[slot], agg.at[didx.at[chunk]],
                              sem_s).wait()

    for b in range(AHEAD):
        _fire(b, b)
    plsc.subcore_barrier()

    def _group(g, carry):
        i0 = g * KSLOT
        for b in range(KSLOT):
            chunk = i0 + b
            _wait_gather(b, chunk)
            _scatter(b, chunk)
            prev = chunk - AHEAD

            @pl.when(prev >= 0)
            def _():
                _wait_scatter((b - AHEAD) % KSLOT, prev)
            nxt = chunk + AHEAD

            @pl.when(nxt < CH_PER_W)
            def _():
                _fire((b + AHEAD) % KSLOT, nxt)
        return carry

    lax.fori_loop(0, CH_PER_W // KSLOT, _group, 0)

    for b in range(AHEAD):
        chunk = CH_PER_W - AHEAD + b
        _wait_scatter(chunk % KSLOT, chunk)

    plsc.subcore_barrier()
    pltpu.sync_copy(agg.at[pl.ds(s * ROWS_PER_TILE, ROWS_PER_TILE)],
                    out_hbm.at[c, pl.ds(s * ROWS_PER_TILE, ROWS_PER_TILE)])


_sc_agg = pl.kernel(
    _sc_agg_body,
    out_type=jax.ShapeDtypeStruct((NC, N_PAD, DP), jnp.float32),
    mesh=plsc.VectorSubcoreMesh(core_axis_name="c", subcore_axis_name="s"),
    scratch_types=[
        pltpu.VMEM((CH_PER_W, CHUNK), jnp.int32),
        pltpu.VMEM((CH_PER_W, CHUNK), jnp.int32),
        pltpu.VMEM((KSLOT, CHUNK, DP), jnp.float32),
        pltpu.VMEM_SHARED((N_PAD, DP), jnp.float32),
        pltpu.SemaphoreType.DMA,
        pltpu.SemaphoreType.DMA,
        pltpu.SemaphoreType.DMA,
    ],
    compiler_params=pltpu.CompilerParams(use_tc_tiling_on_sc=False),
)


def _tc_a_body(x_ref, w1l_ref, w1r_ref, p1_ref, r1_ref):
    x = x_ref[...]
    xw = jnp.dot(x, w1l_ref[...], preferred_element_type=jnp.float32)
    col = lax.broadcasted_iota(jnp.int32, (N, DP), 1)
    p1_ref[...] = jnp.where(col == ONES_COL, 1.0, xw)
    r1_ref[...] = jnp.dot(x, w1r_ref[...], preferred_element_type=jnp.float32)


def _tc_b_body(parts_ref, r1_ref, b1_ref, w2l_ref, w2r_ref, e36_ref,
               p2_ref, r2_ref, deg_ref):
    agg = parts_ref[0, :N] + parts_ref[1, :N]
    deg = jnp.maximum(
        jnp.dot(agg, e36_ref[...], preferred_element_type=jnp.float32), 1.0)
    h = jnp.maximum(agg / deg + r1_ref[...] + b1_ref[...], 0.0)
    p2_ref[...] = jnp.dot(h, w2l_ref[...], preferred_element_type=jnp.float32)
    r2_ref[...] = jnp.dot(h, w2r_ref[...], preferred_element_type=jnp.float32)
    deg_ref[...] = deg


def _tc_c_body(parts_ref, r2_ref, deg_ref, b2_ref, out_ref):
    agg = parts_ref[0, :N] + parts_ref[1, :N]
    out_ref[...] = agg / deg_ref[...] + r2_ref[...] + b2_ref[...]


_tc_a = pl.pallas_call(
    _tc_a_body,
    out_shape=[jax.ShapeDtypeStruct((N, DP), jnp.float32),
               jax.ShapeDtypeStruct((N, DP), jnp.float32)],
)

_tc_b = pl.pallas_call(
    _tc_b_body,
    out_shape=[jax.ShapeDtypeStruct((N, DP), jnp.float32),
               jax.ShapeDtypeStruct((N, DP), jnp.float32),
               jax.ShapeDtypeStruct((N, 1), jnp.float32)],
)

_tc_c = pl.pallas_call(
    _tc_c_body,
    out_shape=jax.ShapeDtypeStruct((N, DP), jnp.float32),
)


def kernel(x, edge_index, W1_l, b1, W1_r, W2_l, b2, W2_r):
    src = edge_index[0].reshape(NCHUNKS, CHUNK)
    dst = edge_index[1].reshape(NCHUNKS, CHUNK)

    w1l = jnp.zeros((F_IN, DP), jnp.float32).at[:, :36].set(W1_l)
    w1r = jnp.zeros((F_IN, DP), jnp.float32).at[:, :36].set(W1_r)
    b1p = jnp.zeros((1, DP), jnp.float32).at[0, :36].set(b1)
    w2l = jnp.zeros((DP, DP), jnp.float32).at[:36, :40].set(W2_l)
    w2r = jnp.zeros((DP, DP), jnp.float32).at[:36, :40].set(W2_r)
    b2p = jnp.zeros((1, DP), jnp.float32).at[0, :40].set(b2)
    e36 = jnp.zeros((DP, 1), jnp.float32).at[ONES_COL, 0].set(1.0)
    zeros = jnp.zeros((N_PAD, DP), jnp.float32)

    p1, r1 = _tc_a(x, w1l, w1r)
    parts1 = _sc_agg(p1, src, dst, zeros)
    p2, r2, deg = _tc_b(parts1, r1, b1p, w2l, w2r, e36)
    parts2 = _sc_agg(p2, src, dst, zeros)
    return _tc_c(parts2, r2, deg, b2p)
